# Initial kernel scaffold; baseline (speedup 1.0000x reference)
import jax
import jax.numpy as jnp
from jax.experimental import pallas as pl


def kernel(pred_tensor, target_tensor):
    raise NotImplementedError("write your pallas kernel here")


# speedup vs baseline: 3.5961x; 3.5961x over previous
import functools

import jax
import jax.numpy as jnp
from jax import lax
from jax.experimental import pallas as pl
from jax.experimental.pallas import tpu as pltpu
from jax.experimental.pallas import tpu_sc as plsc

_C = 12544
_F = 30
_NW = 32
_PER_W = _C // _NW
_NCH = (_PER_W + 15) // 16
_INV_N = 1.0 / 64.0

_mesh = plsc.VectorSubcoreMesh(core_axis_name="c", subcore_axis_name="s")


def _sqrt16(x):
    i = plsc.bitcast(x, jnp.int32)
    y = plsc.bitcast((i >> 1) + 0x1FBD1DF5, jnp.float32)
    y = 0.5 * (y + x / y)
    y = 0.5 * (y + x / y)
    y = 0.5 * (y + x / y)
    return y


@functools.partial(
    pl.kernel,
    out_type=jax.ShapeDtypeStruct((_NW, 16), jnp.float32),
    mesh=_mesh,
    compiler_params=pltpu.CompilerParams(needs_layout_passes=False),
    scratch_types=[
        pltpu.VMEM((_PER_W * _F,), jnp.float32),
        pltpu.VMEM((_PER_W * _F,), jnp.float32),
        pltpu.VMEM_SHARED((16, 16), jnp.float32),
        pltpu.VMEM((16, 16), jnp.float32),
        pltpu.VMEM((16,), jnp.float32),
        pltpu.SemaphoreType.DMA,
        pltpu.SemaphoreType.DMA,
    ],
)
def _yolo_sc(pred_hbm, targ_hbm, out_hbm, pred_v, targ_v, part_sh, red_v,
             vec_v, sem_p, sem_t):
    cid = lax.axis_index("c")
    sid = lax.axis_index("s")
    wid = sid * 2 + cid
    base = wid * (_PER_W * _F)
    cp_p = pltpu.async_copy(pred_hbm.at[pl.ds(base, _PER_W * _F)], pred_v, sem_p)
    cp_t = pltpu.async_copy(targ_hbm.at[pl.ds(base, _PER_W * _F)], targ_v, sem_t)
    cp_p.wait()
    cp_t.wait()

    def chunk(j, acc):
        rows = j * 16 + lax.iota(jnp.int32, 16)
        valid = rows < _PER_W
        ib = jnp.minimum(rows, _PER_W - 1) * _F

        def gp(c):
            return plsc.load_gather(pred_v, [ib + c])

        def gt(c):
            return plsc.load_gather(targ_v, [ib + c])

        px0, py0, pw0, ph0, pc0 = gp(0), gp(1), gp(2), gp(3), gp(4)
        px1, py1, pw1, ph1, pc1 = gp(5), gp(6), gp(7), gp(8), gp(9)
        tx0, ty0, tw0, th0, tcf = gt(0), gt(1), gt(2), gt(3), gt(4)
        tx1, ty1, tw1, th1, tc9 = gt(5), gt(6), gt(7), gt(8), gt(9)

        coo = tcf > 0.0
        noo = tcf == 0.0
        dn0 = pc0 - tcf
        dn1 = pc1 - tc9
        noo_term = dn0 * dn0 + dn1 * dn1

        bx1 = tx0 / 14.0 - 0.5 * tw0
        bx2 = tx0 / 14.0 + 0.5 * tw0
        by1 = ty0 / 14.0 - 0.5 * th0
        by2 = ty0 / 14.0 + 0.5 * th0
        tarea = (bx2 - bx1) * (by2 - by1)

        def iou_of(px, py, pw, ph):
            x1 = px / 14.0 - 0.5 * pw
            x2 = px / 14.0 + 0.5 * pw
            y1 = py / 14.0 - 0.5 * ph
            y2 = py / 14.0 + 0.5 * ph
            wx = jnp.maximum(jnp.minimum(x2, bx2) - jnp.maximum(x1, bx1), 0.0)
            wy = jnp.maximum(jnp.minimum(y2, by2) - jnp.maximum(y1, by1), 0.0)
            inter = wx * wy
            area = (x2 - x1) * (y2 - y1)
            return inter / (area + tarea - inter)

        iou0 = iou_of(px0, py0, pw0, ph0)
        iou1 = iou_of(px1, py1, pw1, ph1)
        sel = iou1 > iou0
        max_iou = jnp.maximum(iou0, iou1)

        rx = jnp.where(sel, px1, px0)
        ry = jnp.where(sel, py1, py0)
        rw = jnp.where(sel, pw1, pw0)
        rh = jnp.where(sel, ph1, ph0)
        rc = jnp.where(sel, pc1, pc0)
        nc = jnp.where(sel, pc0, pc1)
        trx = jnp.where(sel, tx1, tx0)
        try_ = jnp.where(sel, ty1, ty0)
        trw = jnp.where(sel, tw1, tw0)
        trh = jnp.where(sel, th1, th0)

        dx = rx - trx
        dy = ry - try_
        loc = (dx * dx + dy * dy
               + (rw + trw - 2.0 * _sqrt16(rw * trw))
               + (rh + trh - 2.0 * _sqrt16(rh * trh)))
        dcon = rc - max_iou

        cls = jnp.zeros((16,), jnp.float32)
        for c in range(10, 30):
            dcl = gp(c) - gt(c)
            cls = cls + dcl * dcl

        obj_term = 5.0 * loc + 2.0 * (dcon * dcon) + nc * nc + cls
        cell = (jnp.where(coo, obj_term, 0.0)
                + 0.5 * jnp.where(noo, noo_term, 0.0))
        return acc + jnp.where(valid, cell, 0.0)

    acc = lax.fori_loop(0, _NCH, chunk, jnp.zeros((16,), jnp.float32))

    vec_v[...] = acc * _INV_N
    pltpu.sync_copy(vec_v, out_hbm.at[wid])


def kernel(pred_tensor, target_tensor):
    parts = _yolo_sc(pred_tensor.reshape(-1), target_tensor.reshape(-1))
    return jnp.sum(parts)


# speedup vs baseline: 6.2519x; 1.7385x over previous
import functools

import jax
import jax.numpy as jnp
from jax import lax
from jax.experimental import pallas as pl
from jax.experimental.pallas import tpu as pltpu
from jax.experimental.pallas import tpu_sc as plsc

_NPOS = 196
_B = 64
_F = 30
_PLANE = _F * _B
_NW = 32
_NCHT = _NPOS * 4
_WIN = 8 * _PLANE
_ANCH_MAX = _NPOS - 8
_INV_N = 1.0 / 64.0

_mesh = plsc.VectorSubcoreMesh(core_axis_name="c", subcore_axis_name="s")


def _sqrt16(x):
    i = plsc.bitcast(x, jnp.int32)
    y = plsc.bitcast((i >> 1) + 0x1FBD1DF5, jnp.float32)
    y = 0.5 * (y + x / y)
    y = 0.5 * (y + x / y)
    y = 0.5 * (y + x / y)
    return y


@functools.partial(
    pl.kernel,
    out_type=jax.ShapeDtypeStruct((_NW, 16), jnp.float32),
    mesh=_mesh,
    compiler_params=pltpu.CompilerParams(needs_layout_passes=False),
    scratch_types=[
        pltpu.VMEM((_WIN,), jnp.float32),
        pltpu.VMEM((_WIN,), jnp.float32),
        pltpu.VMEM((16,), jnp.float32),
        pltpu.SemaphoreType.DMA,
        pltpu.SemaphoreType.DMA,
    ],
)
def _yolo_sc(pred_hbm, targ_hbm, out_hbm, pred_v, targ_v, vec_v, sem_p, sem_t):
    cid = lax.axis_index("c")
    sid = lax.axis_index("s")
    wid = sid * 2 + cid
    k0 = (_NCHT * wid + (_NW - 1)) // _NW
    k1 = (_NCHT * (wid + 1) + (_NW - 1)) // _NW
    nch = k1 - k0
    anchor = jnp.minimum(k0 // 4, _ANCH_MAX)
    base = anchor * _PLANE
    cp_p = pltpu.async_copy(pred_hbm.at[pl.ds(base, _WIN)], pred_v, sem_p)
    cp_t = pltpu.async_copy(targ_hbm.at[pl.ds(base, _WIN)], targ_v, sem_t)
    cp_p.wait()
    cp_t.wait()

    def chunk(j, acc):
        k = k0 + j
        lbase = (k // 4 - anchor) * _PLANE + (k % 4) * 16

        def gp(c):
            return pred_v[pl.ds(lbase + c * _B, 16)]

        def gt(c):
            return targ_v[pl.ds(lbase + c * _B, 16)]

        px0, py0, pw0, ph0, pc0 = gp(0), gp(1), gp(2), gp(3), gp(4)
        px1, py1, pw1, ph1, pc1 = gp(5), gp(6), gp(7), gp(8), gp(9)
        tx0, ty0, tw0, th0, tcf = gt(0), gt(1), gt(2), gt(3), gt(4)
        tx1, ty1, tw1, th1, tc9 = gt(5), gt(6), gt(7), gt(8), gt(9)

        coo = tcf > 0.0
        noo = tcf == 0.0
        dn0 = pc0 - tcf
        dn1 = pc1 - tc9
        noo_term = dn0 * dn0 + dn1 * dn1

        bx1 = tx0 / 14.0 - 0.5 * tw0
        bx2 = tx0 / 14.0 + 0.5 * tw0
        by1 = ty0 / 14.0 - 0.5 * th0
        by2 = ty0 / 14.0 + 0.5 * th0
        tarea = (bx2 - bx1) * (by2 - by1)

        def iou_of(px, py, pw, ph):
            x1 = px / 14.0 - 0.5 * pw
            x2 = px / 14.0 + 0.5 * pw
            y1 = py / 14.0 - 0.5 * ph
            y2 = py / 14.0 + 0.5 * ph
            wx = jnp.maximum(jnp.minimum(x2, bx2) - jnp.maximum(x1, bx1), 0.0)
            wy = jnp.maximum(jnp.minimum(y2, by2) - jnp.maximum(y1, by1), 0.0)
            inter = wx * wy
            area = (x2 - x1) * (y2 - y1)
            return inter / (area + tarea - inter)

        iou0 = iou_of(px0, py0, pw0, ph0)
        iou1 = iou_of(px1, py1, pw1, ph1)
        sel = iou1 > iou0
        max_iou = jnp.maximum(iou0, iou1)

        rx = jnp.where(sel, px1, px0)
        ry = jnp.where(sel, py1, py0)
        rw = jnp.where(sel, pw1, pw0)
        rh = jnp.where(sel, ph1, ph0)
        rc = jnp.where(sel, pc1, pc0)
        nc = jnp.where(sel, pc0, pc1)
        trx = jnp.where(sel, tx1, tx0)
        try_ = jnp.where(sel, ty1, ty0)
        trw = jnp.where(sel, tw1, tw0)
        trh = jnp.where(sel, th1, th0)

        dx = rx - trx
        dy = ry - try_
        loc = (dx * dx + dy * dy
               + (rw + trw - 2.0 * _sqrt16(rw * trw))
               + (rh + trh - 2.0 * _sqrt16(rh * trh)))
        dcon = rc - max_iou

        cls = jnp.zeros((16,), jnp.float32)
        for c in range(10, 30):
            dcl = gp(c) - gt(c)
            cls = cls + dcl * dcl

        obj_term = 5.0 * loc + 2.0 * (dcon * dcon) + nc * nc + cls
        cell = (jnp.where(coo, obj_term, 0.0)
                + 0.5 * jnp.where(noo, noo_term, 0.0))
        return acc + cell

    acc = lax.fori_loop(0, nch, chunk, jnp.zeros((16,), jnp.float32))

    vec_v[...] = acc * _INV_N
    pltpu.sync_copy(vec_v, out_hbm.at[wid])


def kernel(pred_tensor, target_tensor):
    pf = jnp.transpose(pred_tensor, (1, 2, 3, 0)).reshape(-1)
    tf = jnp.transpose(target_tensor, (1, 2, 3, 0)).reshape(-1)
    parts = _yolo_sc(pf, tf)
    return jnp.sum(parts)


# speedup vs baseline: 6.4103x; 1.0253x over previous
import functools

import jax
import jax.numpy as jnp
from jax import lax
from jax.experimental import pallas as pl
from jax.experimental.pallas import tpu as pltpu
from jax.experimental.pallas import tpu_sc as plsc

_NPOS = 196
_B = 64
_F = 30
_PLANE = _F * _B
_NW = 32
_NCHT = _NPOS * 4
_WIN = 8 * _PLANE
_ANCH_MAX = _NPOS - 8
_INV_N = 1.0 / 64.0

_mesh = plsc.VectorSubcoreMesh(core_axis_name="c", subcore_axis_name="s")


def _sqrt16(x):
    i = plsc.bitcast(x, jnp.int32)
    r = plsc.bitcast(0x5F3759DF - (i >> 1), jnp.float32)
    hx = 0.5 * x
    r = r * (1.5 - hx * r * r)
    r = r * (1.5 - hx * r * r)
    return x * r


@functools.partial(
    pl.kernel,
    out_type=jax.ShapeDtypeStruct((_NW, 16), jnp.float32),
    mesh=_mesh,
    compiler_params=pltpu.CompilerParams(needs_layout_passes=False),
    scratch_types=[
        pltpu.VMEM((_WIN,), jnp.float32),
        pltpu.VMEM((_WIN,), jnp.float32),
        pltpu.VMEM((16,), jnp.float32),
        pltpu.SemaphoreType.DMA,
        pltpu.SemaphoreType.DMA,
    ],
)
def _yolo_sc(pred_hbm, targ_hbm, out_hbm, pred_v, targ_v, vec_v, sem_p, sem_t):
    cid = lax.axis_index("c")
    sid = lax.axis_index("s")
    wid = sid * 2 + cid
    k0 = (_NCHT * wid + (_NW - 1)) // _NW
    k1 = (_NCHT * (wid + 1) + (_NW - 1)) // _NW
    nch = k1 - k0
    anchor = jnp.minimum(k0 // 4, _ANCH_MAX)
    base = anchor * _PLANE
    cp_p = pltpu.async_copy(pred_hbm.at[pl.ds(base, _WIN)], pred_v, sem_p)
    cp_t = pltpu.async_copy(targ_hbm.at[pl.ds(base, _WIN)], targ_v, sem_t)
    cp_p.wait()
    cp_t.wait()

    def chunk(j, acc):
        k = k0 + j
        lbase = (k // 4 - anchor) * _PLANE + (k % 4) * 16

        def gp(c):
            return pred_v[pl.ds(lbase + c * _B, 16)]

        def gt(c):
            return targ_v[pl.ds(lbase + c * _B, 16)]

        px0, py0, pw0, ph0, pc0 = gp(0), gp(1), gp(2), gp(3), gp(4)
        px1, py1, pw1, ph1, pc1 = gp(5), gp(6), gp(7), gp(8), gp(9)
        tx0, ty0, tw0, th0, tcf = gt(0), gt(1), gt(2), gt(3), gt(4)
        tx1, ty1, tw1, th1, tc9 = gt(5), gt(6), gt(7), gt(8), gt(9)

        coo = tcf > 0.0
        noo = tcf == 0.0
        dn0 = pc0 - tcf
        dn1 = pc1 - tc9
        noo_term = dn0 * dn0 + dn1 * dn1

        inv14 = jnp.float32(1.0 / 14.0)
        bx1 = tx0 * inv14 - 0.5 * tw0
        bx2 = tx0 * inv14 + 0.5 * tw0
        by1 = ty0 * inv14 - 0.5 * th0
        by2 = ty0 * inv14 + 0.5 * th0
        tarea = (bx2 - bx1) * (by2 - by1)

        def iou_of(px, py, pw, ph):
            x1 = px * inv14 - 0.5 * pw
            x2 = px * inv14 + 0.5 * pw
            y1 = py * inv14 - 0.5 * ph
            y2 = py * inv14 + 0.5 * ph
            wx = jnp.maximum(jnp.minimum(x2, bx2) - jnp.maximum(x1, bx1), 0.0)
            wy = jnp.maximum(jnp.minimum(y2, by2) - jnp.maximum(y1, by1), 0.0)
            inter = wx * wy
            area = (x2 - x1) * (y2 - y1)
            return inter / (area + tarea - inter)

        iou0 = iou_of(px0, py0, pw0, ph0)
        iou1 = iou_of(px1, py1, pw1, ph1)
        sel = iou1 > iou0
        max_iou = jnp.maximum(iou0, iou1)

        rx = jnp.where(sel, px1, px0)
        ry = jnp.where(sel, py1, py0)
        rw = jnp.where(sel, pw1, pw0)
        rh = jnp.where(sel, ph1, ph0)
        rc = jnp.where(sel, pc1, pc0)
        nc = jnp.where(sel, pc0, pc1)
        trx = jnp.where(sel, tx1, tx0)
        try_ = jnp.where(sel, ty1, ty0)
        trw = jnp.where(sel, tw1, tw0)
        trh = jnp.where(sel, th1, th0)

        dx = rx - trx
        dy = ry - try_
        loc = (dx * dx + dy * dy
               + (rw + trw - 2.0 * _sqrt16(rw * trw))
               + (rh + trh - 2.0 * _sqrt16(rh * trh)))
        dcon = rc - max_iou

        cls = jnp.zeros((16,), jnp.float32)
        for c in range(10, 30):
            dcl = gp(c) - gt(c)
            cls = cls + dcl * dcl

        obj_term = 5.0 * loc + 2.0 * (dcon * dcon) + nc * nc + cls
        cell = (jnp.where(coo, obj_term, 0.0)
                + 0.5 * jnp.where(noo, noo_term, 0.0))
        return acc + cell

    acc = lax.fori_loop(0, nch, chunk, jnp.zeros((16,), jnp.float32))

    vec_v[...] = acc * _INV_N
    pltpu.sync_copy(vec_v, out_hbm.at[wid])


def kernel(pred_tensor, target_tensor):
    pf = jnp.transpose(pred_tensor, (1, 2, 3, 0)).reshape(-1)
    tf = jnp.transpose(target_tensor, (1, 2, 3, 0)).reshape(-1)
    parts = _yolo_sc(pf, tf)
    return jnp.sum(parts)
